# baseline (speedup 1.0000x reference)
import jax
import jax.numpy as jnp
from jax import lax
from jax.experimental import pallas as pl
from jax.experimental.pallas import tpu as pltpu
from jax.experimental.pallas import tpu_sc as plsc

NUM_NODES = 10000
FEAT = 128
HIDDEN = 128
B = 512
S1 = 10
S2 = 25
NC, NS, LANES = 2, 16, 16
NW = NC * NS
BPW = B // NW
R1 = BPW * S1
SEG_PER_CHUNK = 5
ROWS_PER_CHUNK = 128
NCHUNK = R1 // SEG_PER_CHUNK
NB = 3
FCH = FEAT // LANES
ADJ_PAD = 128
SREG = R1 + SEG_PER_CHUNK


def _sc_body(feat_hbm, adj_hbm, batch_hbm,
             h0_hbm, h1_hbm, nm1_hbm,
             bidx, adj0, adj1, s2buf, sidxbuf, h1rows, f0rows, fbuf,
             shared, sem0, sem1,
             gsem0, gsem1, gsem2, ssem0, ssem1, ssem2):
    gsems = (gsem0, gsem1, gsem2)
    ssems = (ssem0, ssem1, ssem2)
    wid = lax.axis_index("s") * NC + lax.axis_index("c")
    sid = lax.axis_index("s")
    base = pl.multiple_of(wid * BPW, 8)
    sbase = sid * SREG
    iot = lax.iota(jnp.int32, LANES)

    pltpu.sync_copy(batch_hbm.at[pl.ds(base, BPW)], bidx)
    pltpu.async_copy(adj_hbm.at[bidx], adj0, sem0).wait()

    for r in range(BPW):
        pltpu.async_copy(feat_hbm.at[adj0.at[r, pl.ds(0, S1)]],
                         h1rows.at[pl.ds(r * S1, S1)], sem1)
        pltpu.async_copy(adj_hbm.at[adj0.at[r, pl.ds(0, S1)]],
                         adj1.at[pl.ds(r * S1, S1)], sem0)
    pltpu.async_copy(feat_hbm.at[bidx], f0rows, sem1)

    def zbody(row, carry):
        for cc in range(FCH):
            fbuf[0, row, pl.ds(cc * LANES, LANES)] = jnp.zeros((LANES,),
                                                               jnp.float32)
        return carry

    lax.fori_loop(0, ROWS_PER_CHUNK, zbody, 0)
    pltpu.sync_copy(fbuf.at[0],
                    shared.at[pl.ds(sbase, ROWS_PER_CHUNK)])
    pltpu.sync_copy(fbuf.at[0, pl.ds(0, SREG - ROWS_PER_CHUNK)],
                    shared.at[pl.ds(sbase + ROWS_PER_CHUNK,
                                    SREG - ROWS_PER_CHUNK)])

    for r in range(BPW):
        pltpu.make_async_copy(adj_hbm.at[adj0.at[r, pl.ds(0, S1)]],
                              adj1.at[pl.ds(r * S1, S1)], sem0).wait()
    for r in range(BPW):
        pltpu.make_async_copy(feat_hbm.at[adj0.at[r, pl.ds(0, S1)]],
                              h1rows.at[pl.ds(r * S1, S1)], sem1).wait()
    pltpu.make_async_copy(feat_hbm.at[bidx], f0rows, sem1).wait()
    pltpu.sync_copy(h1rows, h1_hbm.at[pl.ds(pl.multiple_of(wid * R1, 8), R1)])
    pltpu.sync_copy(f0rows, h0_hbm.at[pl.ds(base, BPW)])

    def build_gidx(k, b):
        for s in range(SEG_PER_CHUNK):
            r = k * SEG_PER_CHUNK + s
            s2buf[b, pl.ds(s * S2, LANES)] = adj1[r, pl.ds(0, LANES)]
            s2buf[b, pl.ds(s * S2 + (S2 - LANES), LANES)] = (
                adj1[r, pl.ds(S2 - LANES, LANES)])

    def build_sidx(k, b):
        for c in range(ROWS_PER_CHUNK // LANES):
            j = c * LANES + iot
            seg = lax.div(j, S2)
            vals = jnp.where(j < SEG_PER_CHUNK * S2,
                             sbase + k * SEG_PER_CHUNK + seg,
                             sbase + R1 + (c % SEG_PER_CHUNK))
            sidxbuf[b, pl.ds(c * LANES, LANES)] = vals

    def issue_chunk(b, sem):
        pltpu.async_copy(feat_hbm.at[s2buf.at[b, pl.ds(0, ROWS_PER_CHUNK)]],
                         fbuf.at[b], sem)

    def wait_chunk(b, sem):
        pltpu.make_async_copy(
            feat_hbm.at[s2buf.at[b, pl.ds(0, ROWS_PER_CHUNK)]],
            fbuf.at[b], sem).wait()

    for b in range(NB):
        s2buf[b, pl.ds(ROWS_PER_CHUNK - LANES, LANES)] = jnp.zeros(
            (LANES,), jnp.int32)

    def chunkbody(kk, carry):
        for b in range(NB):
            k = kk * NB + b

            @pl.when(k < NCHUNK)
            def _():
                @pl.when(k >= NB)
                def _():
                    pltpu.make_async_copy(fbuf.at[b],
                                          shared.at[sidxbuf.at[b]],
                                          ssems[b]).wait()
                build_gidx(k, b)
                issue_chunk(b, gsems[b])

            kp = k - (NB - 1)
            bp = (b + 1) % NB

            @pl.when((kp >= 0) & (kp < NCHUNK))
            def _():
                wait_chunk(bp, gsems[bp])
                build_sidx(kp, bp)
                pltpu.async_copy(fbuf.at[bp], shared.at[sidxbuf.at[bp]],
                                 ssems[bp], add=True)
        return carry

    lax.fori_loop(0, (NCHUNK + NB - 1 + (NB - 1)) // NB, chunkbody, 0)
    for b in range(NB):
        pltpu.make_async_copy(fbuf.at[b], shared.at[sidxbuf.at[b]],
                              ssems[b]).wait()
    pltpu.sync_copy(shared.at[pl.ds(sbase, R1)],
                    nm1_hbm.at[pl.ds(pl.multiple_of(wid * R1, 8), R1)])


def _sc_gather(features, adj32, batch32):
    f32 = jnp.float32
    kfn = pl.kernel(
        _sc_body,
        out_type=[
            jax.ShapeDtypeStruct((B, FEAT), f32),
            jax.ShapeDtypeStruct((B * S1, FEAT), f32),
            jax.ShapeDtypeStruct((B * S1, FEAT), f32),
        ],
        mesh=plsc.VectorSubcoreMesh(core_axis_name="c", subcore_axis_name="s",
                                    num_cores=NC, num_subcores=NS),
        scratch_types=[
            pltpu.VMEM((BPW,), jnp.int32),
            pltpu.VMEM((BPW, ADJ_PAD), jnp.int32),
            pltpu.VMEM((R1, ADJ_PAD), jnp.int32),
            pltpu.VMEM((NB, ROWS_PER_CHUNK), jnp.int32),
            pltpu.VMEM((NB, ROWS_PER_CHUNK), jnp.int32),
            pltpu.VMEM((R1, FEAT), f32),
            pltpu.VMEM((BPW, FEAT), f32),
            pltpu.VMEM((NB, ROWS_PER_CHUNK, FEAT), f32),
            pltpu.VMEM_SHARED((NS * SREG, FEAT), f32),
        ] + [pltpu.SemaphoreType.DMA] * 8,
    )
    return kfn(features, adj32, batch32)


TCBLK = 640
TCGRID = B * S1 // TCBLK
GPB = TCBLK // S1


def _tc_body(h1g, nm1, h0g, w0bd1, w0bd0, wn1, ws1, out_ref, macc, nacc):
    i = pl.program_id(0)
    x = jnp.concatenate([h1g[...], nm1[...]], axis=1)
    h = jnp.maximum(jnp.dot(x, w0bd1[...], preferred_element_type=jnp.float32),
                    0.0)
    pblk = jnp.dot(h, wn1[...], preferred_element_type=jnp.float32)
    macc[pl.ds(i * GPB, GPB), :] = jnp.sum(
        pblk.reshape(GPB, S1, HIDDEN), axis=1)
    nacc[pl.ds(i * GPB, GPB), :] = jnp.sum(
        h1g[...].reshape(GPB, S1, FEAT), axis=1)

    @pl.when(i == TCGRID - 1)
    def _():
        x0 = jnp.concatenate([h0g[...], nacc[...]], axis=1)
        h0 = jnp.maximum(
            jnp.dot(x0, w0bd0[...], preferred_element_type=jnp.float32), 0.0)
        out = jnp.concatenate(
            [jnp.dot(h0, ws1[...], preferred_element_type=jnp.float32),
             macc[...] * (1.0 / S1)], axis=1)
        nrm = jnp.sqrt(jnp.sum(out * out, axis=1, keepdims=True))
        out_ref[...] = out / jnp.maximum(nrm, 1e-12)


def _blockdiag(wa, wb):
    z = jnp.zeros((HIDDEN, HIDDEN), jnp.float32)
    return jnp.concatenate(
        [jnp.concatenate([wa, z], axis=1),
         jnp.concatenate([z, wb], axis=1)], axis=0)


def kernel(features, w_self_0, w_neigh_0, w_self_1, w_neigh_1, adj, batch):
    adj32 = jnp.pad(adj.astype(jnp.int32), ((0, 0), (0, ADJ_PAD - 32)))
    batch32 = batch.astype(jnp.int32)
    h0g, h1g, nm1s = _sc_gather(features, adj32, batch32)

    w0bd1 = _blockdiag(w_self_0, w_neigh_0 * (1.0 / S2))
    w0bd0 = _blockdiag(w_self_0, w_neigh_0 * (1.0 / S1))

    out = pl.pallas_call(
        _tc_body,
        grid=(TCGRID,),
        in_specs=[
            pl.BlockSpec((TCBLK, FEAT), lambda i: (i, 0)),
            pl.BlockSpec((TCBLK, FEAT), lambda i: (i, 0)),
            pl.BlockSpec((B, FEAT), lambda i: (0, 0)),
            pl.BlockSpec((2 * HIDDEN, 2 * HIDDEN), lambda i: (0, 0)),
            pl.BlockSpec((2 * HIDDEN, 2 * HIDDEN), lambda i: (0, 0)),
            pl.BlockSpec((2 * HIDDEN, HIDDEN), lambda i: (0, 0)),
            pl.BlockSpec((2 * HIDDEN, HIDDEN), lambda i: (0, 0)),
        ],
        out_specs=pl.BlockSpec((B, 2 * HIDDEN), lambda i: (0, 0)),
        out_shape=jax.ShapeDtypeStruct((B, 2 * HIDDEN), jnp.float32),
        scratch_shapes=[
            pltpu.VMEM((B, HIDDEN), jnp.float32),
            pltpu.VMEM((B, FEAT), jnp.float32),
        ],
    )(h1g, nm1s, h0g, w0bd1, w0bd0, w_neigh_1, w_self_1)
    return out


# speedup vs baseline: 2.2391x; 2.2391x over previous
import jax
import jax.numpy as jnp
from jax import lax
from jax.experimental import pallas as pl
from jax.experimental.pallas import tpu as pltpu
from jax.experimental.pallas import tpu_sc as plsc

NUM_NODES = 10000
FEAT = 128
HIDDEN = 128
B = 512
S1 = 10
S2 = 25
NC, NS, LANES = 2, 16, 16
NW = NC * NS
BPW = B // NW
R1 = BPW * S1
SEG_PER_CHUNK = 4
SEG_STRIDE = 32
ROWS_PER_CHUNK = 128
NCHUNK = R1 // SEG_PER_CHUNK
NB = 3
FCH = FEAT // LANES
ADJ_PAD = 128
SREG = R1 + SEG_PER_CHUNK


def _sc_body(feat_hbm, adj_hbm, batch_hbm,
             h0_hbm, h1_hbm, nm1_hbm,
             bidx, adj0, adj1, s2buf, sidxbuf, h1rows, f0rows, fbuf,
             shared, sem0, sem1,
             gsem0, gsem1, gsem2, ssem0, ssem1, ssem2):
    gsems = (gsem0, gsem1, gsem2)
    ssems = (ssem0, ssem1, ssem2)
    wid = lax.axis_index("s") * NC + lax.axis_index("c")
    sid = lax.axis_index("s")
    base = pl.multiple_of(wid * BPW, 8)
    sbase = sid * SREG
    iot = lax.iota(jnp.int32, LANES)

    pltpu.sync_copy(batch_hbm.at[pl.ds(base, BPW)], bidx)
    pltpu.async_copy(adj_hbm.at[bidx], adj0, sem0).wait()

    for r in range(BPW):
        pltpu.async_copy(feat_hbm.at[adj0.at[r, pl.ds(0, S1)]],
                         h1rows.at[pl.ds(r * S1, S1)], sem1)
        pltpu.async_copy(adj_hbm.at[adj0.at[r, pl.ds(0, S1)]],
                         adj1.at[pl.ds(r * S1, S1)], sem0)
    pltpu.async_copy(feat_hbm.at[bidx], f0rows, sem1)

    def zbody(row, carry):
        for cc in range(FCH):
            fbuf[0, row, pl.ds(cc * LANES, LANES)] = jnp.zeros((LANES,),
                                                               jnp.float32)
        return carry

    lax.fori_loop(0, ROWS_PER_CHUNK, zbody, 0)
    pltpu.sync_copy(fbuf.at[0],
                    shared.at[pl.ds(sbase, ROWS_PER_CHUNK)])
    pltpu.sync_copy(fbuf.at[0, pl.ds(0, SREG - ROWS_PER_CHUNK)],
                    shared.at[pl.ds(sbase + ROWS_PER_CHUNK,
                                    SREG - ROWS_PER_CHUNK)])

    for r in range(BPW):
        pltpu.make_async_copy(adj_hbm.at[adj0.at[r, pl.ds(0, S1)]],
                              adj1.at[pl.ds(r * S1, S1)], sem0).wait()
    for r in range(BPW):
        pltpu.make_async_copy(feat_hbm.at[adj0.at[r, pl.ds(0, S1)]],
                              h1rows.at[pl.ds(r * S1, S1)], sem1).wait()
    pltpu.make_async_copy(feat_hbm.at[bidx], f0rows, sem1).wait()
    pltpu.sync_copy(h1rows, h1_hbm.at[pl.ds(pl.multiple_of(wid * R1, 8), R1)])
    pltpu.sync_copy(f0rows, h0_hbm.at[pl.ds(base, BPW)])

    def build_gidx(k, b):
        for s in range(SEG_PER_CHUNK):
            r = k * SEG_PER_CHUNK + s
            s2buf[b, pl.ds(s * SEG_STRIDE, LANES)] = adj1[r, pl.ds(0, LANES)]
            s2buf[b, pl.ds(s * SEG_STRIDE + LANES, LANES)] = (
                adj1[r, pl.ds(LANES, LANES)])

    def build_sidx(k, b):
        for c in range(2 * SEG_PER_CHUNK):
            t = k * SEG_PER_CHUNK + c // 2
            if c % 2 == 0:
                vals = jnp.zeros((LANES,), jnp.int32) + (sbase + t)
            else:
                trash = sbase + R1 + (c // 2)
                vals = jnp.where(iot < (S2 - LANES), sbase + t, trash)
            sidxbuf[b, pl.ds(c * LANES, LANES)] = vals

    def issue_chunk(b, sem):
        pltpu.async_copy(feat_hbm.at[s2buf.at[b, pl.ds(0, ROWS_PER_CHUNK)]],
                         fbuf.at[b], sem)

    def wait_chunk(b, sem):
        pltpu.make_async_copy(
            feat_hbm.at[s2buf.at[b, pl.ds(0, ROWS_PER_CHUNK)]],
            fbuf.at[b], sem).wait()

    def chunkbody(kk, carry):
        for b in range(NB):
            k = kk * NB + b

            @pl.when(k < NCHUNK)
            def _():
                @pl.when(k >= NB)
                def _():
                    pltpu.make_async_copy(fbuf.at[b],
                                          shared.at[sidxbuf.at[b]],
                                          ssems[b]).wait()
                build_gidx(k, b)
                issue_chunk(b, gsems[b])

            kp = k - (NB - 1)
            bp = (b + 1) % NB

            @pl.when((kp >= 0) & (kp < NCHUNK))
            def _():
                wait_chunk(bp, gsems[bp])
                build_sidx(kp, bp)
                pltpu.async_copy(fbuf.at[bp], shared.at[sidxbuf.at[bp]],
                                 ssems[bp], add=True)
        return carry

    lax.fori_loop(0, (NCHUNK + NB - 1 + (NB - 1)) // NB, chunkbody, 0)
    for b in range(NB):
        pltpu.make_async_copy(fbuf.at[b], shared.at[sidxbuf.at[b]],
                              ssems[b]).wait()
    pltpu.sync_copy(shared.at[pl.ds(sbase, R1)],
                    nm1_hbm.at[pl.ds(pl.multiple_of(wid * R1, 8), R1)])


def _sc_gather(features, adj32, batch32):
    f32 = jnp.float32
    kfn = pl.kernel(
        _sc_body,
        out_type=[
            jax.ShapeDtypeStruct((B, FEAT), f32),
            jax.ShapeDtypeStruct((B * S1, FEAT), f32),
            jax.ShapeDtypeStruct((B * S1, FEAT), f32),
        ],
        mesh=plsc.VectorSubcoreMesh(core_axis_name="c", subcore_axis_name="s",
                                    num_cores=NC, num_subcores=NS),
        scratch_types=[
            pltpu.VMEM((BPW,), jnp.int32),
            pltpu.VMEM((BPW, ADJ_PAD), jnp.int32),
            pltpu.VMEM((R1, ADJ_PAD), jnp.int32),
            pltpu.VMEM((NB, ROWS_PER_CHUNK), jnp.int32),
            pltpu.VMEM((NB, ROWS_PER_CHUNK), jnp.int32),
            pltpu.VMEM((R1, FEAT), f32),
            pltpu.VMEM((BPW, FEAT), f32),
            pltpu.VMEM((NB, ROWS_PER_CHUNK, FEAT), f32),
            pltpu.VMEM_SHARED((NS * SREG, FEAT), f32),
        ] + [pltpu.SemaphoreType.DMA] * 8,
    )
    return kfn(features, adj32, batch32)


TCBLK = 1280
TCGRID = B * S1 // TCBLK
GPB = TCBLK // S1


def _tc_body(h1g, nm1, h0g, w0bd1, w0bd0, wn1, ws1, out_ref, macc, nacc):
    i = pl.program_id(0)
    x = jnp.concatenate([h1g[...], nm1[...]], axis=1)
    h = jnp.maximum(jnp.dot(x, w0bd1[...], preferred_element_type=jnp.float32),
                    0.0)
    pblk = jnp.dot(h, wn1[...], preferred_element_type=jnp.float32)
    macc[pl.ds(i * GPB, GPB), :] = jnp.sum(
        pblk.reshape(GPB, S1, HIDDEN), axis=1)
    nacc[pl.ds(i * GPB, GPB), :] = jnp.sum(
        h1g[...].reshape(GPB, S1, FEAT), axis=1)

    @pl.when(i == TCGRID - 1)
    def _():
        x0 = jnp.concatenate([h0g[...], nacc[...]], axis=1)
        h0 = jnp.maximum(
            jnp.dot(x0, w0bd0[...], preferred_element_type=jnp.float32), 0.0)
        out = jnp.concatenate(
            [jnp.dot(h0, ws1[...], preferred_element_type=jnp.float32),
             macc[...] * (1.0 / S1)], axis=1)
        nrm = jnp.sqrt(jnp.sum(out * out, axis=1, keepdims=True))
        out_ref[...] = out / jnp.maximum(nrm, 1e-12)


def _blockdiag(wa, wb):
    z = jnp.zeros((HIDDEN, HIDDEN), jnp.float32)
    return jnp.concatenate(
        [jnp.concatenate([wa, z], axis=1),
         jnp.concatenate([z, wb], axis=1)], axis=0)


def kernel(features, w_self_0, w_neigh_0, w_self_1, w_neigh_1, adj, batch):
    adj32 = jnp.pad(adj.astype(jnp.int32), ((0, 0), (0, ADJ_PAD - 32)))
    batch32 = batch.astype(jnp.int32)
    h0g, h1g, nm1s = _sc_gather(features, adj32, batch32)

    w0bd1 = _blockdiag(w_self_0, w_neigh_0 * (1.0 / S2))
    w0bd0 = _blockdiag(w_self_0, w_neigh_0 * (1.0 / S1))

    out = pl.pallas_call(
        _tc_body,
        grid=(TCGRID,),
        in_specs=[
            pl.BlockSpec((TCBLK, FEAT), lambda i: (i, 0)),
            pl.BlockSpec((TCBLK, FEAT), lambda i: (i, 0)),
            pl.BlockSpec((B, FEAT), lambda i: (0, 0)),
            pl.BlockSpec((2 * HIDDEN, 2 * HIDDEN), lambda i: (0, 0)),
            pl.BlockSpec((2 * HIDDEN, 2 * HIDDEN), lambda i: (0, 0)),
            pl.BlockSpec((2 * HIDDEN, HIDDEN), lambda i: (0, 0)),
            pl.BlockSpec((2 * HIDDEN, HIDDEN), lambda i: (0, 0)),
        ],
        out_specs=pl.BlockSpec((B, 2 * HIDDEN), lambda i: (0, 0)),
        out_shape=jax.ShapeDtypeStruct((B, 2 * HIDDEN), jnp.float32),
        scratch_shapes=[
            pltpu.VMEM((B, HIDDEN), jnp.float32),
            pltpu.VMEM((B, FEAT), jnp.float32),
        ],
    )(h1g, nm1s, h0g, w0bd1, w0bd0, w_neigh_1, w_self_1)
    return out


# speedup vs baseline: 2.2735x; 1.0154x over previous
import jax
import jax.numpy as jnp
from jax import lax
from jax.experimental import pallas as pl
from jax.experimental.pallas import tpu as pltpu
from jax.experimental.pallas import tpu_sc as plsc

NUM_NODES = 10000
FEAT = 128
HIDDEN = 128
B = 512
S1 = 10
S2 = 25
NC, NS, LANES = 2, 16, 16
NW = NC * NS
BPW = B // NW
R1 = BPW * S1
SEG_PER_CHUNK = 4
SEG_STRIDE = 32
ROWS_PER_CHUNK = 128
NCHUNK = R1 // SEG_PER_CHUNK
NB = 3
FCH = FEAT // LANES
ADJ_PAD = 128
SREG = R1 + SEG_PER_CHUNK


def _sc_body(feat_hbm, adj_hbm, batch_hbm,
             h0_hbm, h1_hbm, nm1_hbm,
             bidx, adj0, adj1, s2buf, sidxbuf, h1rows, f0rows, fbuf,
             shared, sem0, sem1,
             gsem0, gsem1, gsem2, ssem0, ssem1, ssem2):
    gsems = (gsem0, gsem1, gsem2)
    ssems = (ssem0, ssem1, ssem2)
    wid = lax.axis_index("s") * NC + lax.axis_index("c")
    sid = lax.axis_index("s")
    base = pl.multiple_of(wid * BPW, 8)
    sbase = sid * SREG
    iot = lax.iota(jnp.int32, LANES)

    pltpu.sync_copy(batch_hbm.at[pl.ds(base, BPW)], bidx)
    pltpu.async_copy(adj_hbm.at[bidx], adj0, sem0).wait()

    for r in range(BPW):
        pltpu.async_copy(feat_hbm.at[adj0.at[r, pl.ds(0, S1)]],
                         h1rows.at[pl.ds(r * S1, S1)], sem1)
        pltpu.async_copy(adj_hbm.at[adj0.at[r, pl.ds(0, S1)]],
                         adj1.at[pl.ds(r * S1, S1)], sem0)
    pltpu.async_copy(feat_hbm.at[bidx], f0rows, sem1)

    def zbody(row, carry):
        for cc in range(FCH):
            fbuf[0, row, pl.ds(cc * LANES, LANES)] = jnp.zeros((LANES,),
                                                               jnp.float32)
        return carry

    lax.fori_loop(0, ROWS_PER_CHUNK, zbody, 0)
    pltpu.sync_copy(fbuf.at[0],
                    shared.at[pl.ds(sbase, ROWS_PER_CHUNK)])
    pltpu.sync_copy(fbuf.at[0, pl.ds(0, SREG - ROWS_PER_CHUNK)],
                    shared.at[pl.ds(sbase + ROWS_PER_CHUNK,
                                    SREG - ROWS_PER_CHUNK)])

    for r in range(BPW):
        pltpu.make_async_copy(adj_hbm.at[adj0.at[r, pl.ds(0, S1)]],
                              adj1.at[pl.ds(r * S1, S1)], sem0).wait()

    def build_gidx(k, b):
        for s in range(SEG_PER_CHUNK):
            r = k * SEG_PER_CHUNK + s
            s2buf[b, pl.ds(s * SEG_STRIDE, LANES)] = adj1[r, pl.ds(0, LANES)]
            s2buf[b, pl.ds(s * SEG_STRIDE + LANES, LANES)] = (
                adj1[r, pl.ds(LANES, LANES)])

    def build_sidx(k, b):
        for c in range(2 * SEG_PER_CHUNK):
            t = k * SEG_PER_CHUNK + c // 2
            if c % 2 == 0:
                vals = jnp.zeros((LANES,), jnp.int32) + (sbase + t)
            else:
                trash = sbase + R1 + (c // 2)
                vals = jnp.where(iot < (S2 - LANES), sbase + t, trash)
            sidxbuf[b, pl.ds(c * LANES, LANES)] = vals

    def issue_chunk(b, sem):
        pltpu.async_copy(feat_hbm.at[s2buf.at[b, pl.ds(0, ROWS_PER_CHUNK)]],
                         fbuf.at[b], sem)

    def wait_chunk(b, sem):
        pltpu.make_async_copy(
            feat_hbm.at[s2buf.at[b, pl.ds(0, ROWS_PER_CHUNK)]],
            fbuf.at[b], sem).wait()

    def chunkbody(kk, carry):
        for b in range(NB):
            k = kk * NB + b

            @pl.when(k < NCHUNK)
            def _():
                @pl.when(k >= NB)
                def _():
                    pltpu.make_async_copy(fbuf.at[b],
                                          shared.at[sidxbuf.at[b]],
                                          ssems[b]).wait()
                build_gidx(k, b)
                issue_chunk(b, gsems[b])

            kp = k - (NB - 1)
            bp = (b + 1) % NB

            @pl.when((kp >= 0) & (kp < NCHUNK))
            def _():
                wait_chunk(bp, gsems[bp])
                build_sidx(kp, bp)
                pltpu.async_copy(fbuf.at[bp], shared.at[sidxbuf.at[bp]],
                                 ssems[bp], add=True)
        return carry

    lax.fori_loop(0, (NCHUNK + NB - 1 + (NB - 1)) // NB, chunkbody, 0)
    for r in range(BPW):
        pltpu.make_async_copy(feat_hbm.at[adj0.at[r, pl.ds(0, S1)]],
                              h1rows.at[pl.ds(r * S1, S1)], sem1).wait()
    pltpu.make_async_copy(feat_hbm.at[bidx], f0rows, sem1).wait()
    pltpu.sync_copy(h1rows, h1_hbm.at[pl.ds(pl.multiple_of(wid * R1, 8), R1)])
    pltpu.sync_copy(f0rows, h0_hbm.at[pl.ds(base, BPW)])
    for b in range(NB):
        pltpu.make_async_copy(fbuf.at[b], shared.at[sidxbuf.at[b]],
                              ssems[b]).wait()
    pltpu.sync_copy(shared.at[pl.ds(sbase, R1)],
                    nm1_hbm.at[pl.ds(pl.multiple_of(wid * R1, 8), R1)])


def _sc_gather(features, adj32, batch32):
    f32 = jnp.float32
    kfn = pl.kernel(
        _sc_body,
        out_type=[
            jax.ShapeDtypeStruct((B, FEAT), f32),
            jax.ShapeDtypeStruct((B * S1, FEAT), f32),
            jax.ShapeDtypeStruct((B * S1, FEAT), f32),
        ],
        mesh=plsc.VectorSubcoreMesh(core_axis_name="c", subcore_axis_name="s",
                                    num_cores=NC, num_subcores=NS),
        scratch_types=[
            pltpu.VMEM((BPW,), jnp.int32),
            pltpu.VMEM((BPW, ADJ_PAD), jnp.int32),
            pltpu.VMEM((R1, ADJ_PAD), jnp.int32),
            pltpu.VMEM((NB, ROWS_PER_CHUNK), jnp.int32),
            pltpu.VMEM((NB, ROWS_PER_CHUNK), jnp.int32),
            pltpu.VMEM((R1, FEAT), f32),
            pltpu.VMEM((BPW, FEAT), f32),
            pltpu.VMEM((NB, ROWS_PER_CHUNK, FEAT), f32),
            pltpu.VMEM_SHARED((NS * SREG, FEAT), f32),
        ] + [pltpu.SemaphoreType.DMA] * 8,
    )
    return kfn(features, adj32, batch32)


TCBLK = 1280
TCGRID = B * S1 // TCBLK
GPB = TCBLK // S1


def _tc_body(h1g, nm1, h0g, w0bd1, w0bd0, wn1, ws1, out_ref, macc, nacc):
    i = pl.program_id(0)
    x = jnp.concatenate([h1g[...], nm1[...]], axis=1)
    h = jnp.maximum(jnp.dot(x, w0bd1[...], preferred_element_type=jnp.float32),
                    0.0)
    pblk = jnp.dot(h, wn1[...], preferred_element_type=jnp.float32)
    macc[pl.ds(i * GPB, GPB), :] = jnp.sum(
        pblk.reshape(GPB, S1, HIDDEN), axis=1)
    nacc[pl.ds(i * GPB, GPB), :] = jnp.sum(
        h1g[...].reshape(GPB, S1, FEAT), axis=1)

    @pl.when(i == TCGRID - 1)
    def _():
        x0 = jnp.concatenate([h0g[...], nacc[...]], axis=1)
        h0 = jnp.maximum(
            jnp.dot(x0, w0bd0[...], preferred_element_type=jnp.float32), 0.0)
        out = jnp.concatenate(
            [jnp.dot(h0, ws1[...], preferred_element_type=jnp.float32),
             macc[...] * (1.0 / S1)], axis=1)
        nrm = jnp.sqrt(jnp.sum(out * out, axis=1, keepdims=True))
        out_ref[...] = out / jnp.maximum(nrm, 1e-12)


def _blockdiag(wa, wb):
    z = jnp.zeros((HIDDEN, HIDDEN), jnp.float32)
    return jnp.concatenate(
        [jnp.concatenate([wa, z], axis=1),
         jnp.concatenate([z, wb], axis=1)], axis=0)


def kernel(features, w_self_0, w_neigh_0, w_self_1, w_neigh_1, adj, batch):
    adj32 = jnp.pad(adj.astype(jnp.int32), ((0, 0), (0, ADJ_PAD - 32)))
    batch32 = batch.astype(jnp.int32)
    h0g, h1g, nm1s = _sc_gather(features, adj32, batch32)

    w0bd1 = _blockdiag(w_self_0, w_neigh_0 * (1.0 / S2))
    w0bd0 = _blockdiag(w_self_0, w_neigh_0 * (1.0 / S1))

    out = pl.pallas_call(
        _tc_body,
        grid=(TCGRID,),
        in_specs=[
            pl.BlockSpec((TCBLK, FEAT), lambda i: (i, 0)),
            pl.BlockSpec((TCBLK, FEAT), lambda i: (i, 0)),
            pl.BlockSpec((B, FEAT), lambda i: (0, 0)),
            pl.BlockSpec((2 * HIDDEN, 2 * HIDDEN), lambda i: (0, 0)),
            pl.BlockSpec((2 * HIDDEN, 2 * HIDDEN), lambda i: (0, 0)),
            pl.BlockSpec((2 * HIDDEN, HIDDEN), lambda i: (0, 0)),
            pl.BlockSpec((2 * HIDDEN, HIDDEN), lambda i: (0, 0)),
        ],
        out_specs=pl.BlockSpec((B, 2 * HIDDEN), lambda i: (0, 0)),
        out_shape=jax.ShapeDtypeStruct((B, 2 * HIDDEN), jnp.float32),
        scratch_shapes=[
            pltpu.VMEM((B, HIDDEN), jnp.float32),
            pltpu.VMEM((B, FEAT), jnp.float32),
        ],
    )(h1g, nm1s, h0g, w0bd1, w0bd0, w_neigh_1, w_self_1)
    return out
